# baseline (speedup 1.0000x reference)
import functools

import jax
import jax.numpy as jnp
from jax import lax
from jax.experimental import pallas as pl
from jax.experimental.pallas import tpu as pltpu
from jax.experimental.pallas import tpu_sc as plsc

_N = 1024
_D = 32
_V = 512
_L = 16


def _gather_body(idx_hbm, table_hbm, out_hbm, idx_v, table_v,
                 rows2, sem, *, n_workers):
    mesh_nc = lax.axis_size("c")
    wid = lax.axis_index("s") * mesh_nc + lax.axis_index("c")
    rows_per_w = _N // n_workers
    base = wid * rows_per_w

    pltpu.sync_copy(table_hbm, table_v)
    pltpu.sync_copy(idx_hbm.at[pl.ds(base // 8, rows_per_w // 8)], idx_v)

    def compute_row(i, rows_v):
        rb = i // 8
        r_in = i % 8

        @plsc.parallel_loop(0, _N // _L)
        def group(g):
            jb = g // 8
            j_in = (g % 8) * _L
            idx_vec = idx_v[rb, jb, r_in, pl.ds(j_in, _L)]
            for d in range(_D):
                vals = plsc.load_gather(table_v, [idx_vec + d * _V])
                rows_v[d // 8, jb, d % 8, pl.ds(j_in, _L)] = vals

    def row(i, carry):
        r = base + i
        buf = i % 2
        rows_v = rows2.at[buf]

        @pl.when(i > 1)
        def _wait():
            pltpu.make_async_copy(rows_v, out_hbm.at[r - 2],
                                  sem.at[buf]).wait()

        compute_row(i, rows_v)
        pltpu.async_copy(rows_v, out_hbm.at[r], sem.at[buf])
        return carry

    lax.fori_loop(0, rows_per_w, row, 0)
    last = base + rows_per_w
    pltpu.make_async_copy(rows2.at[0], out_hbm.at[last - 2], sem.at[0]).wait()
    pltpu.make_async_copy(rows2.at[1], out_hbm.at[last - 1], sem.at[1]).wait()


def kernel(spd_matrix, W):
    mesh = plsc.VectorSubcoreMesh(core_axis_name="c", subcore_axis_name="s")
    nw = mesh.num_cores * mesh.num_subcores
    body = functools.partial(_gather_body, n_workers=nw)
    f = pl.kernel(
        body,
        mesh=mesh,
        compiler_params=pltpu.CompilerParams(
            needs_layout_passes=False, use_tc_tiling_on_sc=False),
        out_type=jax.ShapeDtypeStruct((_N, _D // 8, _N // 128, 8, 128),
                                      jnp.float32),
        scratch_types=[
            pltpu.VMEM((_N // nw // 8, _N // 128, 8, 128), jnp.int32),
            pltpu.VMEM((_D * _V,), jnp.float32),
            pltpu.VMEM((2, _D // 8, _N // 128, 8, 128), jnp.float32),
            pltpu.SemaphoreType.DMA((2,)),
        ],
    )
    spd5 = spd_matrix.reshape(_N // 8, 8, _N // 128, 128)
    spd5 = jnp.transpose(spd5, (0, 2, 1, 3))
    out = f(spd5, W.T.reshape(_D * _V))
    out = jnp.transpose(out, (0, 2, 4, 1, 3))
    return out.reshape(_N, _N, _D)


# speedup vs baseline: 1.0127x; 1.0127x over previous
import functools

import jax
import jax.numpy as jnp
from jax import lax
from jax.experimental import pallas as pl
from jax.experimental.pallas import tpu as pltpu
from jax.experimental.pallas import tpu_sc as plsc

_N = 1024
_D = 32
_V = 512
_L = 16


def _gather_body(idx_hbm, table_hbm, out_hbm, idx_v, table_v,
                 rows2, sem, *, n_workers):
    mesh_nc = lax.axis_size("c")
    wid = lax.axis_index("s") * mesh_nc + lax.axis_index("c")
    rows_per_w = _N // n_workers
    base = wid * rows_per_w

    idx_band = idx_hbm.at[pl.ds(base // 8, rows_per_w // 8)]
    pltpu.async_copy(table_hbm, table_v, sem.at[0])
    pltpu.async_copy(idx_band, idx_v, sem.at[1])
    pltpu.make_async_copy(table_hbm, table_v, sem.at[0]).wait()
    pltpu.make_async_copy(idx_band, idx_v, sem.at[1]).wait()

    def compute_row(i, rows_v):
        rb = i // 8
        r_in = i % 8

        @plsc.parallel_loop(0, _N // _L)
        def group(g):
            jb = g // 8
            j_in = (g % 8) * _L
            idx_vec = idx_v[rb, jb, r_in, pl.ds(j_in, _L)]
            for d in range(_D):
                vals = plsc.load_gather(table_v, [idx_vec + d * _V])
                rows_v[d // 8, jb, d % 8, pl.ds(j_in, _L)] = vals

    def row(i, carry):
        r = base + i
        buf = i % 2
        rows_v = rows2.at[buf]

        @pl.when(i > 1)
        def _wait():
            pltpu.make_async_copy(rows_v, out_hbm.at[r - 2],
                                  sem.at[buf]).wait()

        compute_row(i, rows_v)
        pltpu.async_copy(rows_v, out_hbm.at[r], sem.at[buf])
        return carry

    lax.fori_loop(0, rows_per_w, row, 0)
    last = base + rows_per_w
    pltpu.make_async_copy(rows2.at[0], out_hbm.at[last - 2], sem.at[0]).wait()
    pltpu.make_async_copy(rows2.at[1], out_hbm.at[last - 1], sem.at[1]).wait()


def kernel(spd_matrix, W):
    mesh = plsc.VectorSubcoreMesh(core_axis_name="c", subcore_axis_name="s")
    nw = mesh.num_cores * mesh.num_subcores
    body = functools.partial(_gather_body, n_workers=nw)
    f = pl.kernel(
        body,
        mesh=mesh,
        compiler_params=pltpu.CompilerParams(
            needs_layout_passes=False, use_tc_tiling_on_sc=False),
        out_type=jax.ShapeDtypeStruct((_N, _D // 8, _N // 128, 8, 128),
                                      jnp.float32),
        scratch_types=[
            pltpu.VMEM((_N // nw // 8, _N // 128, 8, 128), jnp.int32),
            pltpu.VMEM((_D * _V,), jnp.float32),
            pltpu.VMEM((2, _D // 8, _N // 128, 8, 128), jnp.float32),
            pltpu.SemaphoreType.DMA((2,)),
        ],
    )
    spd5 = spd_matrix.reshape(_N // 8, 8, _N // 128, 128)
    spd5 = jnp.transpose(spd5, (0, 2, 1, 3))
    out = f(spd5, W.T.reshape(_D * _V))
    out = jnp.transpose(out, (0, 2, 4, 1, 3))
    return out.reshape(_N, _N, _D)
